# baseline (speedup 1.0000x reference)
import functools

import jax
import jax.numpy as jnp
from jax import lax
from jax.experimental import pallas as pl
from jax.experimental.pallas import tpu as pltpu
from jax.experimental.pallas import tpu_sc as plsc

N_ROWS = 1000000
EMB_DIM = 64
BATCH = 4096
LANES = 128

_BN = 512
_NBUF = 6


@functools.cache
def _make_sc_gather():
    info = plsc.get_sparse_core_info()
    nc, ns = info.num_cores, info.num_subcores
    bpw = BATCH // (nc * ns)

    def body(user_hbm, item_hbm, users_hbm, pos_hbm, u_out, i_out,
             uidx_v, iidx_v, uslab, islab, uout_v, iout_v, usem, isem):
        c = lax.axis_index("c")
        s = lax.axis_index("s")
        wid = c * ns + s
        base = wid * bpw
        pltpu.sync_copy(users_hbm.at[pl.ds(base, bpw)], uidx_v.at[pl.ds(0, bpw)])
        pltpu.sync_copy(pos_hbm.at[pl.ds(base, bpw)], iidx_v.at[pl.ds(0, bpw)])

        def sidx(idx_v, gk):
            return idx_v[pl.ds(gk, 16)][0]

        def issue(tab_hbm, idx_v, slab, sem, gk, b):
            start = pl.multiple_of((sidx(idx_v, gk) >> 7) << 7, LANES)
            pltpu.async_copy(tab_hbm.at[:, pl.ds(start, LANES)],
                             slab.at[b], sem.at[b])

        def drain(tab_hbm, slab, sem, b):
            pltpu.make_async_copy(tab_hbm.at[:, pl.ds(0, LANES)],
                                  slab.at[b], sem.at[b]).wait()

        def extract(idx_v, slab, out_v, gk, b):
            lane = sidx(idx_v, gk) & (LANES - 1)
            cols = jnp.full((16,), lane, jnp.int32)
            outc = jnp.full((16,), gk, jnp.int32)
            chunks = []
            ssq = jnp.zeros((16,), jnp.float32)
            for r in range(EMB_DIM // 16):
                rows = lax.iota(jnp.int32, 16) + (16 * r)
                vals = plsc.load_gather(slab.at[b], [rows, cols])
                chunks.append((rows, vals))
                ssq = ssq + vals * vals
            total = jnp.full((16,), jnp.sum(ssq), jnp.float32)
            ybits = 0x5F3759DF - (plsc.bitcast(total, jnp.int32) >> 1)
            y = plsc.bitcast(ybits, jnp.float32)
            y = y * (1.5 - 0.5 * total * y * y)
            y = y * (1.5 - 0.5 * total * y * y)
            y = y * (1.5 - 0.5 * total * y * y)
            for rows, vals in chunks:
                plsc.store_scatter(out_v, [rows, outc], vals * y)

        for b in range(_NBUF):
            issue(user_hbm, uidx_v, uslab, usem, b, b)
            issue(item_hbm, iidx_v, islab, isem, b, b)

        def outer(gk, carry):
            b = lax.rem(gk, _NBUF)
            drain(user_hbm, uslab, usem, b)
            extract(uidx_v, uslab, uout_v, gk, b)
            drain(item_hbm, islab, isem, b)
            extract(iidx_v, islab, iout_v, gk, b)

            @pl.when(gk + _NBUF < bpw)
            def _():
                issue(user_hbm, uidx_v, uslab, usem, gk + _NBUF, b)
                issue(item_hbm, iidx_v, islab, isem, gk + _NBUF, b)
            return carry

        lax.fori_loop(0, bpw, outer, 0)
        pltpu.sync_copy(uout_v, u_out.at[:, pl.ds(base, bpw)])
        pltpu.sync_copy(iout_v, i_out.at[:, pl.ds(base, bpw)])

    return pl.kernel(
        body,
        mesh=plsc.VectorSubcoreMesh(core_axis_name="c", subcore_axis_name="s"),
        compiler_params=pltpu.CompilerParams(needs_layout_passes=False),
        out_type=[
            jax.ShapeDtypeStruct((EMB_DIM, BATCH), jnp.float32),
            jax.ShapeDtypeStruct((EMB_DIM, BATCH), jnp.float32),
        ],
        scratch_types=[
            pltpu.VMEM((bpw + 16,), jnp.int32),
            pltpu.VMEM((bpw + 16,), jnp.int32),
            pltpu.VMEM((_NBUF, EMB_DIM, LANES), jnp.float32),
            pltpu.VMEM((_NBUF, EMB_DIM, LANES), jnp.float32),
            pltpu.VMEM((EMB_DIM, bpw), jnp.float32),
            pltpu.VMEM((EMB_DIM, bpw), jnp.float32),
            pltpu.SemaphoreType.DMA((_NBUF,)),
            pltpu.SemaphoreType.DMA((_NBUF,)),
        ],
    )


def _normalize_bf16(x):
    norm = jnp.sqrt(jnp.sum(x * x, axis=0, keepdims=True))
    return (x / jnp.maximum(norm, 1e-12)).astype(jnp.bfloat16)


def _mm_body(u_ref, i_ref, o_ref, un_scratch):
    j = pl.program_id(0)

    @pl.when(j == 0)
    def _():
        un_scratch[...] = u_ref[...].astype(jnp.bfloat16)

    ib = i_ref[...].astype(jnp.bfloat16)
    o_ref[...] = lax.dot_general(
        un_scratch[...], ib,
        dimension_numbers=(((0,), (0,)), ((), ())),
        preferred_element_type=jnp.float32,
    )


def _tc_score(u_t, i_t):
    grid = (BATCH // _BN,)
    return pl.pallas_call(
        _mm_body,
        grid=grid,
        in_specs=[
            pl.BlockSpec((EMB_DIM, BATCH), lambda j: (0, 0)),
            pl.BlockSpec((EMB_DIM, _BN), lambda j: (0, j)),
        ],
        out_specs=pl.BlockSpec((BATCH, _BN), lambda j: (0, j)),
        out_shape=jax.ShapeDtypeStruct((BATCH, BATCH), jnp.float32),
        scratch_shapes=[pltpu.VMEM((EMB_DIM, BATCH), jnp.bfloat16)],
        compiler_params=pltpu.CompilerParams(
            fuse_transposed_lhs_in_matmul=True),
    )(u_t, i_t)


def kernel(user_embedding, item_embedding, users, pos_items):
    users = users.astype(jnp.int32)
    pos_items = pos_items.astype(jnp.int32)
    user_t = jnp.transpose(user_embedding)
    item_t = jnp.transpose(item_embedding)
    u_t, i_t = _make_sc_gather()(user_t, item_t, users, pos_items)
    return _tc_score(u_t, i_t)


# speedup vs baseline: 1.0189x; 1.0189x over previous
import functools

import jax
import jax.numpy as jnp
from jax import lax
from jax.experimental import pallas as pl
from jax.experimental.pallas import tpu as pltpu
from jax.experimental.pallas import tpu_sc as plsc

N_ROWS = 1000000
EMB_DIM = 64
BATCH = 4096
LANES = 128

_BN = 512
_NBUF = 6


@functools.cache
def _make_sc_gather():
    info = plsc.get_sparse_core_info()
    nc, ns = info.num_cores, info.num_subcores
    bpw = BATCH // (nc * ns)

    def body(user_hbm, item_hbm, users_hbm, pos_hbm, u_out, i_out,
             uidx_v, iidx_v, uslab, islab, uout_v, iout_v, usem, isem):
        c = lax.axis_index("c")
        s = lax.axis_index("s")
        wid = c * ns + s
        base = wid * bpw
        pltpu.sync_copy(users_hbm.at[pl.ds(base, bpw)], uidx_v.at[pl.ds(0, bpw)])
        pltpu.sync_copy(pos_hbm.at[pl.ds(base, bpw)], iidx_v.at[pl.ds(0, bpw)])

        def sidx(idx_v, gk):
            return idx_v[pl.ds(gk, 16)][0]

        def issue(tab_hbm, idx_v, slab, sem, gk, b):
            start = pl.multiple_of((sidx(idx_v, gk) >> 7) << 7, LANES)
            pltpu.async_copy(tab_hbm.at[:, pl.ds(start, LANES)],
                             slab.at[b], sem.at[b])

        def drain(tab_hbm, slab, sem, b):
            pltpu.make_async_copy(tab_hbm.at[:, pl.ds(0, LANES)],
                                  slab.at[b], sem.at[b]).wait()

        def extract(idx_v, slab, out_v, gk, b):
            lane = sidx(idx_v, gk) & (LANES - 1)
            cols = jnp.full((16,), lane, jnp.int32)
            outc = jnp.full((16,), gk, jnp.int32)
            for r in range(EMB_DIM // 16):
                rows = lax.iota(jnp.int32, 16) + (16 * r)
                vals = plsc.load_gather(slab.at[b], [rows, cols])
                plsc.store_scatter(out_v, [rows, outc], vals)

        for b in range(_NBUF):
            issue(user_hbm, uidx_v, uslab, usem, b, b)
            issue(item_hbm, iidx_v, islab, isem, b, b)

        def outer(gk, carry):
            b = lax.rem(gk, _NBUF)
            drain(user_hbm, uslab, usem, b)
            extract(uidx_v, uslab, uout_v, gk, b)
            drain(item_hbm, islab, isem, b)
            extract(iidx_v, islab, iout_v, gk, b)

            @pl.when(gk + _NBUF < bpw)
            def _():
                issue(user_hbm, uidx_v, uslab, usem, gk + _NBUF, b)
                issue(item_hbm, iidx_v, islab, isem, gk + _NBUF, b)
            return carry

        lax.fori_loop(0, bpw, outer, 0)
        pltpu.sync_copy(uout_v, u_out.at[:, pl.ds(base, bpw)])
        pltpu.sync_copy(iout_v, i_out.at[:, pl.ds(base, bpw)])

    return pl.kernel(
        body,
        mesh=plsc.VectorSubcoreMesh(core_axis_name="c", subcore_axis_name="s"),
        compiler_params=pltpu.CompilerParams(needs_layout_passes=False),
        out_type=[
            jax.ShapeDtypeStruct((EMB_DIM, BATCH), jnp.float32),
            jax.ShapeDtypeStruct((EMB_DIM, BATCH), jnp.float32),
        ],
        scratch_types=[
            pltpu.VMEM((bpw + 16,), jnp.int32),
            pltpu.VMEM((bpw + 16,), jnp.int32),
            pltpu.VMEM((_NBUF, EMB_DIM, LANES), jnp.float32),
            pltpu.VMEM((_NBUF, EMB_DIM, LANES), jnp.float32),
            pltpu.VMEM((EMB_DIM, bpw), jnp.float32),
            pltpu.VMEM((EMB_DIM, bpw), jnp.float32),
            pltpu.SemaphoreType.DMA((_NBUF,)),
            pltpu.SemaphoreType.DMA((_NBUF,)),
        ],
    )


def _normalize_bf16(x):
    norm = jnp.sqrt(jnp.sum(x * x, axis=0, keepdims=True))
    return (x / jnp.maximum(norm, 1e-12)).astype(jnp.bfloat16)


def _mm_body(u_ref, i_ref, o_ref, un_scratch):
    j = pl.program_id(0)

    @pl.when(j == 0)
    def _():
        un_scratch[...] = _normalize_bf16(u_ref[...])

    ib = _normalize_bf16(i_ref[...])
    o_ref[...] = lax.dot_general(
        un_scratch[...], ib,
        dimension_numbers=(((0,), (0,)), ((), ())),
        preferred_element_type=jnp.float32,
    )


def _tc_score(u_t, i_t):
    grid = (BATCH // _BN,)
    return pl.pallas_call(
        _mm_body,
        grid=grid,
        in_specs=[
            pl.BlockSpec((EMB_DIM, BATCH), lambda j: (0, 0)),
            pl.BlockSpec((EMB_DIM, _BN), lambda j: (0, j)),
        ],
        out_specs=pl.BlockSpec((BATCH, _BN), lambda j: (0, j)),
        out_shape=jax.ShapeDtypeStruct((BATCH, BATCH), jnp.float32),
        scratch_shapes=[pltpu.VMEM((EMB_DIM, BATCH), jnp.bfloat16)],
        compiler_params=pltpu.CompilerParams(
            fuse_transposed_lhs_in_matmul=True),
    )(u_t, i_t)


def kernel(user_embedding, item_embedding, users, pos_items):
    users = users.astype(jnp.int32)
    pos_items = pos_items.astype(jnp.int32)
    user_t = jnp.transpose(user_embedding)
    item_t = jnp.transpose(item_embedding)
    u_t, i_t = _make_sc_gather()(user_t, item_t, users, pos_items)
    return _tc_score(u_t, i_t)
